# baseline (speedup 1.0000x reference)
import functools

import jax
import jax.numpy as jnp
from jax import lax
from jax.experimental import pallas as pl
from jax.experimental.pallas import tpu as pltpu
from jax.experimental.pallas import tpu_sc as plsc

_NUM_CORES = 2
_NUM_SUBCORES = 16
_NW = _NUM_CORES * _NUM_SUBCORES
_CHUNK = 1024


def _make_gather(V, D, Btot):
    bpw = Btot // _NW
    nch = bpw // _CHUNK
    mesh = plsc.VectorSubcoreMesh(core_axis_name="c", subcore_axis_name="s")

    @functools.partial(
        pl.kernel,
        mesh=mesh,
        out_type=jax.ShapeDtypeStruct((Btot, D), jnp.float32),
        compiler_params=pltpu.CompilerParams(use_tc_tiling_on_sc=False),
        scratch_types=[
            pltpu.VMEM((bpw,), jnp.int32),
            pltpu.VMEM((_CHUNK, D), jnp.float32),
            pltpu.SemaphoreType.DMA,
        ],
    )
    def k(table_hbm, idx_hbm, out_hbm, idx_v, rows_v, sem):
        wid = lax.axis_index("s") * _NUM_CORES + lax.axis_index("c")
        base = wid * bpw
        pltpu.sync_copy(idx_hbm.at[pl.ds(base, bpw)], idx_v)

        def body(g, carry):
            off = g * _CHUNK
            pltpu.async_copy(
                table_hbm.at[idx_v.at[pl.ds(off, _CHUNK)]], rows_v, sem
            ).wait()
            pltpu.sync_copy(rows_v, out_hbm.at[pl.ds(base + off, _CHUNK)])
            return carry

        lax.fori_loop(0, nch, body, 0)

    return k


def kernel(sequence, table):
    B, L = sequence.shape
    V, D = table.shape
    idx = sequence.reshape(-1).astype(jnp.int32)
    out = _make_gather(V, D, B * L)(table, idx)
    return out.reshape(B, L, D)


# speedup vs baseline: 1.0051x; 1.0051x over previous
import functools

import jax
import jax.numpy as jnp
from jax import lax
from jax.experimental import pallas as pl
from jax.experimental.pallas import tpu as pltpu
from jax.experimental.pallas import tpu_sc as plsc

_NUM_CORES = 2
_NUM_SUBCORES = 16
_NW = _NUM_CORES * _NUM_SUBCORES
_BANK = 512
_SUB = 128
_NSUB = _BANK // _SUB


def _make_gather(V, D, Btot):
    bpw = Btot // _NW
    nrounds = bpw // _BANK
    assert nrounds % 2 == 0
    mesh = plsc.VectorSubcoreMesh(core_axis_name="c", subcore_axis_name="s")

    @functools.partial(
        pl.kernel,
        mesh=mesh,
        out_type=jax.ShapeDtypeStruct((Btot, D), jnp.float32),
        compiler_params=pltpu.CompilerParams(use_tc_tiling_on_sc=False),
        scratch_types=[
            pltpu.VMEM((bpw,), jnp.int32),
            pltpu.VMEM((_BANK, D), jnp.float32),
            pltpu.VMEM((_BANK, D), jnp.float32),
            pltpu.SemaphoreType.DMA,
            pltpu.SemaphoreType.DMA,
            pltpu.SemaphoreType.DMA,
            pltpu.SemaphoreType.DMA,
        ],
    )
    def k(table_hbm, idx_hbm, out_hbm, idx_v, rows0, rows1, g0, g1, s0, s1):
        wid = lax.axis_index("s") * _NUM_CORES + lax.axis_index("c")
        base = wid * bpw
        pltpu.sync_copy(idx_hbm.at[pl.ds(base, bpw)], idx_v)

        banks = ((rows0, g0, s0), (rows1, g1, s1))

        def body(i, carry):
            r0 = i * 2
            descs = []
            for kb in range(2):
                rowsb, gs, ss = banks[kb]
                r = r0 + kb

                @pl.when(i > 0)
                def _():
                    pltpu.make_async_copy(
                        rowsb, out_hbm.at[pl.ds(base, _BANK)], ss
                    ).wait()

                cps = []
                for j in range(_NSUB):
                    off = r * _BANK + j * _SUB
                    cps.append(
                        pltpu.async_copy(
                            table_hbm.at[idx_v.at[pl.ds(off, _SUB)]],
                            rowsb.at[pl.ds(j * _SUB, _SUB)],
                            gs,
                        )
                    )
                descs.append(cps)
            for kb in range(2):
                rowsb, gs, ss = banks[kb]
                r = r0 + kb
                for d in descs[kb]:
                    d.wait()
                pltpu.async_copy(
                    rowsb, out_hbm.at[pl.ds(base + r * _BANK, _BANK)], ss
                )
            return carry

        lax.fori_loop(0, nrounds // 2, body, 0)
        for rowsb, gs, ss in banks:
            pltpu.make_async_copy(
                rowsb, out_hbm.at[pl.ds(base, _BANK)], ss
            ).wait()

    return k


def kernel(sequence, table):
    B, L = sequence.shape
    V, D = table.shape
    idx = sequence.reshape(-1).astype(jnp.int32)
    out = _make_gather(V, D, B * L)(table, idx)
    return out.reshape(B, L, D)


# speedup vs baseline: 1.0887x; 1.0832x over previous
import functools

import jax
import jax.numpy as jnp
from jax import lax
from jax.experimental import pallas as pl
from jax.experimental.pallas import tpu as pltpu
from jax.experimental.pallas import tpu_sc as plsc

_NUM_CORES = 2
_NUM_SUBCORES = 16
_NW = _NUM_CORES * _NUM_SUBCORES
_BANK = 512
_SUB = 128
_NSUB = _BANK // _SUB


def _make_gather(V, D, Btot):
    bpw = Btot // _NW
    nrounds = bpw // _BANK
    assert nrounds % 2 == 0
    mesh = plsc.VectorSubcoreMesh(core_axis_name="c", subcore_axis_name="s")

    @functools.partial(
        pl.kernel,
        mesh=mesh,
        out_type=jax.ShapeDtypeStruct((Btot, D), jnp.float32),
        compiler_params=pltpu.CompilerParams(use_tc_tiling_on_sc=False),
        scratch_types=[
            pltpu.VMEM((bpw,), jnp.int32),
            pltpu.VMEM((_BANK, D), jnp.float32),
            pltpu.VMEM((_BANK, D), jnp.float32),
            pltpu.SemaphoreType.DMA,
            pltpu.SemaphoreType.DMA,
            pltpu.SemaphoreType.DMA,
            pltpu.SemaphoreType.DMA,
        ],
    )
    def k(table_hbm, idx_hbm, out_hbm, idx_v, rows0, rows1, g0, g1, s0, s1):
        wid = lax.axis_index("s") * _NUM_CORES + lax.axis_index("c")
        base = wid * bpw
        pltpu.sync_copy(idx_hbm.at[pl.ds(base, bpw)], idx_v)

        banks = ((rows0, g0, s0), (rows1, g1, s1))

        def body(i, carry):
            r0 = i * 2
            descs = []
            for kb in range(2):
                rowsb, gs, ss = banks[kb]
                r = r0 + kb

                @pl.when(i > 0)
                def _():
                    pltpu.make_async_copy(
                        rowsb, out_hbm.at[pl.ds(base, _BANK)], ss
                    ).wait()

                cps = []
                for j in range(_NSUB):
                    off = r * _BANK + j * _SUB
                    cps.append(
                        pltpu.async_copy(
                            table_hbm.at[idx_v.at[pl.ds(off, _SUB)]],
                            rowsb.at[pl.ds(j * _SUB, _SUB)],
                            gs,
                        )
                    )
                descs.append(cps)
            for kb in range(2):
                rowsb, gs, ss = banks[kb]
                r = r0 + kb
                for d in descs[kb]:
                    d.wait()
                pltpu.async_copy(
                    rowsb, out_hbm.at[pl.ds(base + r * _BANK, _BANK)], ss
                )
            return carry

        lax.fori_loop(0, nrounds // 2, body, 0)
        for rowsb, gs, ss in banks:
            pltpu.make_async_copy(
                rowsb, out_hbm.at[pl.ds(base, _BANK)], ss
            ).wait()

    return k


def kernel(sequence, table):
    B, L = sequence.shape
    V, D = table.shape
    idx = sequence.T.reshape(-1).astype(jnp.int32) * 2
    table_p = jnp.pad(table, ((0, 0), (0, 128 - D))).reshape(2 * V, D)
    out = _make_gather(2 * V, D, B * L)(table_p, idx)
    return out.reshape(L, B, D).transpose(1, 0, 2)
